# baseline (speedup 1.0000x reference)
import functools

import jax
import jax.numpy as jnp
from jax import lax
from jax.experimental import pallas as pl
from jax.experimental.pallas import tpu as pltpu
from jax.experimental.pallas import tpu_sc as plsc

NC, NS = 2, 16
LANES = 16
NCHUNK = 16
CPS = NCHUNK // NC
OCT = 8
CW = 65536
VOCAB = 1000000
CAP = 256
ROWS = 32768
RPW = 1024
D = 64


def _gather_body(table_t, idx, out2, idx_v, widx, glist, glist2, tmp, asm,
                 gsem, ssem, csem, strip):
    c = lax.axis_index("c")
    s = lax.axis_index("s")
    iota = lax.iota(jnp.int32, LANES)
    pltpu.sync_copy(idx.at[s], idx_v.at[pl.ds(0, 8)])
    pltpu.sync_copy(idx.at[s + 16], idx_v.at[pl.ds(8, 8)])

    def cwidth(kk):
        return VOCAB - 15 * CW if kk == CPS - 1 else CW

    def stage(kk, r, p):
        if kk < CPS - 1:
            @pl.when(s < 8)
            def _():
                k = c * CPS + kk
                pltpu.async_copy(
                    table_t.at[8 * r + s, pl.ds(k * CW, CW)],
                    strip.at[p, s, pl.ds(0, CW)],
                    ssem,
                )
        else:
            @pl.when((c == 0) & (s < 8))
            def _():
                pltpu.async_copy(
                    table_t.at[8 * r + s, pl.ds((CPS - 1) * CW, CW)],
                    strip.at[p, s, pl.ds(0, CW)],
                    ssem,
                )
            w15 = VOCAB - 15 * CW
            @pl.when((c == 1) & (s < 8))
            def _():
                pltpu.async_copy(
                    table_t.at[8 * r + s, pl.ds(15 * CW, w15)],
                    strip.at[p, s, pl.ds(0, w15)],
                    ssem,
                )

    def stage_wait(kk):
        if kk < CPS - 1:
            @pl.when(s < 8)
            def _():
                pltpu.make_async_copy(
                    table_t.at[0, pl.ds(0, CW)],
                    strip.at[0, 0, pl.ds(0, CW)],
                    ssem,
                ).wait()
        else:
            @pl.when((c == 0) & (s < 8))
            def _():
                pltpu.make_async_copy(
                    table_t.at[0, pl.ds(0, CW)],
                    strip.at[0, 0, pl.ds(0, CW)],
                    ssem,
                ).wait()
            w15 = VOCAB - 15 * CW
            @pl.when((c == 1) & (s < 8))
            def _():
                pltpu.make_async_copy(
                    table_t.at[0, pl.ds(0, w15)],
                    strip.at[0, 0, pl.ds(0, w15)],
                    ssem,
                ).wait()

    stage(0, 0, 0)
    stage_wait(0)
    plsc.subcore_barrier()

    for kk in range(CPS):
        k = c * CPS + kk
        for g in range(CAP // LANES):
            sl = pl.ds(g * LANES, LANES)
            widx[sl] = iota * 0
            glist[sl] = iota * 0 + ROWS

        def comp(g, off):
            row = g // 8
            col = (g % 8) * LANES
            tvec = idx_v[row, pl.ds(col, LANES)]
            mask = (tvec >> 16) == k
            n = plsc.all_reduce_population_count(mask)
            il = g * LANES + iota
            grow = s * RPW + il + jnp.where(il >= RPW, 15 * RPW, 0)
            offc = jnp.minimum(off, CAP - LANES)
            plsc.store_compressed(
                widx.at[pl.ds(offc, LANES)], tvec & (CW - 1), mask=mask)
            plsc.store_compressed(
                glist.at[pl.ds(offc, LANES)], grow, mask=mask)
            return off + n[0]

        lax.fori_loop(0, 2 * RPW // LANES, comp, 0)

        def gather_assemble(p, col_base):
            copies = [
                pltpu.async_copy(
                    strip.at[p, j].at[widx.at[pl.ds(h * 128, 128)]],
                    tmp.at[pl.ds((j * 2 + h) * 128, 128)],
                    gsem,
                )
                for j in range(OCT)
                for h in range(2)
            ]
            for cp in copies:
                cp.wait()

            def assemble(g, carry):
                for j in range(OCT):
                    vals = tmp[pl.ds(j * 256 + g * LANES, LANES)]
                    plsc.store_scatter(
                        asm, [g * LANES + iota, iota * 0 + (col_base + j)],
                        vals)
                return carry

            lax.fori_loop(0, CAP // LANES, assemble, 0)

        def rbody(r, carry):
            p = r % 2
            stage(kk, r + 1, 1 - p)
            gather_assemble(p, 8 * r)
            stage_wait(kk)
            plsc.subcore_barrier()
            return carry

        lax.fori_loop(0, OCT - 1, rbody, 0)
        if kk + 1 < CPS:
            stage(kk + 1, 0, 0)
        gather_assemble(1, 8 * (OCT - 1))
        if kk + 1 < CPS:
            stage_wait(kk + 1)
        plsc.subcore_barrier()

        for h in range(2):
            for g in range(128 // LANES):
                sl = pl.ds(g * LANES, LANES)
                glist2[h, sl] = glist[pl.ds(h * 128 + g * LANES, LANES)]
        scs = [
            pltpu.async_copy(
                asm.at[pl.ds(h * 128, 128)], out2.at[glist2.at[h]], csem)
            for h in range(2)
        ]
        for cp in scs:
            cp.wait()


def _pos_body(out2, pos, out, in_v, pos_v, out_v, isem, psem):
    wid = lax.axis_index("s") * NC + lax.axis_index("c")
    base = wid * RPW
    p0 = (wid % 2) * RPW
    for sb in range(RPW // 128):
        pcopy = pltpu.async_copy(pos.at[pl.ds(p0 + sb * 128, 128)], pos_v,
                                 psem)
        pltpu.async_copy(out2.at[pl.ds(base + sb * 128, 128)], in_v,
                         isem).wait()
        pcopy.wait()

        def add_row(i, carry):
            for q in range(D // LANES):
                sl = pl.ds(q * LANES, LANES)
                out_v[i, sl] = in_v[i, sl] + pos_v[i, sl]
            return carry

        lax.fori_loop(0, 128, add_row, 0)
        pltpu.sync_copy(out_v, out.at[pl.ds(base + sb * 128, 128)])


@jax.jit
def _embed(idx3, table_t, pos):
    mesh = plsc.VectorSubcoreMesh(
        core_axis_name="c", subcore_axis_name="s", num_cores=NC, num_subcores=NS
    )
    g = pl.kernel(
        _gather_body,
        out_type=jax.ShapeDtypeStruct((ROWS + 1, D), jnp.float32),
        mesh=mesh,
        scratch_types=[
            pltpu.VMEM((16, 128), jnp.int32),
            pltpu.VMEM((CAP,), jnp.int32),
            pltpu.VMEM((CAP,), jnp.int32),
            pltpu.VMEM((2, 128), jnp.int32),
            pltpu.VMEM((2048,), jnp.float32),
            pltpu.VMEM((CAP, D), jnp.float32),
            pltpu.SemaphoreType.DMA,
            pltpu.SemaphoreType.DMA,
            pltpu.SemaphoreType.DMA,
            pltpu.VMEM_SHARED((2, 8, CW), jnp.float32),
        ],
        compiler_params=pltpu.CompilerParams(use_tc_tiling_on_sc=False, needs_layout_passes=False),
    )
    out2 = g(table_t, idx3)
    f = pl.kernel(
        _pos_body,
        out_type=jax.ShapeDtypeStruct((ROWS, D), jnp.float32),
        mesh=mesh,
        scratch_types=[
            pltpu.VMEM((128, D), jnp.float32),
            pltpu.VMEM((128, D), jnp.float32),
            pltpu.VMEM((128, D), jnp.float32),
            pltpu.SemaphoreType.DMA,
            pltpu.SemaphoreType.DMA,
        ],
        compiler_params=pltpu.CompilerParams(use_tc_tiling_on_sc=False, needs_layout_passes=False),
    )
    return f(out2[:ROWS], pos)


def kernel(token_indices, token_table, pos_table):
    b, s = token_indices.shape
    v, d = token_table.shape
    assert b * s == ROWS and d == D and v == VOCAB
    idx3 = token_indices.astype(jnp.int32).reshape(32, 8, 128)
    out = _embed(idx3, token_table.T, pos_table)
    return out.reshape(b, s, d)
